# Initial kernel scaffold; baseline (speedup 1.0000x reference)
import jax
import jax.numpy as jnp
from jax.experimental import pallas as pl


def kernel(codes):
    raise NotImplementedError("write your pallas kernel here")


# speedup vs baseline: 77.0932x; 77.0932x over previous
import jax
import jax.numpy as jnp
import numpy as np
from jax.experimental import pallas as pl
from jax.experimental.pallas import tpu as pltpu

_FACTORS = (4, 4, 16, 5, 3, 5, 5, 6, 7, 4)
_ADD = tuple(np.concatenate([[0], np.cumsum(_FACTORS)[:-1]]).tolist())
_NCH = sum(_FACTORS)
_K = 165
_BM = 64


def _decode_kernel(codes_ref, out_ref):
    q = codes_ref[...].astype(jnp.float32)
    lo = jnp.zeros(q.shape, jnp.int32)
    hi = jnp.zeros(q.shape, jnp.int32)
    for c, f in enumerate(_FACTORS):
        if f & (f - 1) == 0:
            qn = jnp.floor(q * (1.0 / f))
        else:
            qn = jnp.floor((q + 0.5) * (1.0 / f))
        d = q - f * qn
        q = qn
        pos = d.astype(jnp.int32)
        if _ADD[c] + f <= 32:
            lo = lo | (1 << (pos + _ADD[c]))
        else:
            hi = hi | (1 << (pos + (_ADD[c] - 32)))
    m = jnp.concatenate(
        [
            jnp.broadcast_to(lo[:, None, :], (_BM, 32, _K)),
            jnp.broadcast_to(hi[:, None, :], (_BM, _NCH - 32, _K)),
        ],
        axis=1,
    )
    shamt = jax.lax.broadcasted_iota(jnp.int32, (1, _NCH, 1), 1) & 31
    out_ref[...] = ((m >> shamt) & 1).astype(jnp.float32)


@jax.jit
def kernel(codes):
    batch = codes.shape[0]
    grid = (batch // _BM,)
    out = pl.pallas_call(
        _decode_kernel,
        grid=grid,
        in_specs=[pl.BlockSpec((_BM, _K), lambda i: (i, 0))],
        out_specs=pl.BlockSpec((_BM, _NCH, _K), lambda i: (i, 0, 0)),
        out_shape=jax.ShapeDtypeStruct((batch, _NCH, _K), jnp.float32),
        compiler_params=pltpu.CompilerParams(
            dimension_semantics=("parallel",),
        ),
    )(codes)
    return out.reshape(batch, _NCH, 11, 15)
